# baseline (speedup 1.0000x reference)
import functools

import jax
import jax.numpy as jnp
from jax import lax
from jax.experimental import pallas as pl
from jax.experimental.pallas import tpu as pltpu
from jax.experimental.pallas import tpu_sc as plsc

_VOCAB = 100000
_D = 128
_B = 16384
_S = 200
_W = _D // 2

_NC = 2
_NS = 16
_NW = _NC * _NS
_BPW = _B // _NW
_G = 16
_NGRP = _BPW // _G
_NB = 8
_RU = 4
_LANES = 16
_NCH = _D // (2 * _LANES)


def _fire_gather(table_hbm, idx_v, buf, b, sem):
    h0 = pltpu.async_copy(
        table_hbm.at[idx_v.at[pl.ds(b * _S, 128)]],
        buf.at[pl.ds(0, 128)], sem)
    h1 = pltpu.async_copy(
        table_hbm.at[idx_v.at[pl.ds(b * _S + 128, _S - 128)]],
        buf.at[pl.ds(128, _S - 128)], sem)
    return (h0, h1)


def _reduce_mean(buf, out_v, b):
    def body(r, accs):
        accs = list(accs)
        for u in range(_RU // 2):
            row = r * _RU + 2 * u
            for c in range(_NCH):
                w0 = buf[row, pl.ds(c * _LANES, _LANES)]
                w1 = buf[row + 1, pl.ds(c * _LANES, _LANES)]
                s = (plsc.bitcast(w0, jnp.bfloat16)
                     + plsc.bitcast(w1, jnp.bfloat16))
                a, bb = plsc.unpack(s, format=plsc.PackFormat.INTERLEAVED)
                accs[2 * c] = accs[2 * c] + a
                accs[2 * c + 1] = accs[2 * c + 1] + bb
        return tuple(accs)

    zero = jnp.zeros((_LANES,), jnp.float32)
    accs = lax.fori_loop(0, _S // _RU, body, (zero,) * (2 * _NCH))
    inv = jnp.float32(1.0 / _S)
    for c in range(_NCH):
        out_v[b, pl.ds(c * 2 * _LANES, _LANES)] = accs[2 * c] * inv
        out_v[b, pl.ds(c * 2 * _LANES + _LANES, _LANES)] = accs[2 * c + 1] * inv


def _emb_mean_body(ids_hbm, table_hbm, out_hbm,
                   idx0, idx1, rows0, rows1, rows2, rows3, rows4, rows5,
                   rows6, rows7, outv0, outv1,
                   semg0, semg1, semg2, semg3, semg4, semg5, semg6, semg7,
                   semi0, semi1, semo0, semo1):
    wid = lax.axis_index("s") * _NC + lax.axis_index("c")
    base_b = wid * _BPW
    bufs = (rows0, rows1, rows2, rows3, rows4, rows5, rows6, rows7)
    gsems = (semg0, semg1, semg2, semg3, semg4, semg5, semg6, semg7)
    idxs = (idx0, idx1)
    isems = (semi0, semi1)
    outs = (outv0, outv1)
    osems = (semo0, semo1)
    half = _NGRP // 2

    def wait_buf(k):
        pltpu.make_async_copy(table_hbm.at[pl.ds(0, _S)], bufs[k],
                              gsems[k]).wait()

    def wait_idx(p):
        pltpu.make_async_copy(ids_hbm.at[pl.ds(base_b * _S, _G * _S)],
                              idxs[p], isems[p]).wait()

    def drain_out(p):
        pltpu.make_async_copy(outs[p], out_hbm.at[pl.ds(base_b, _G)],
                              osems[p]).wait()

    pltpu.sync_copy(ids_hbm.at[pl.ds(base_b * _S, _G * _S)], idx0)
    for b in range(_NB - 1):
        _fire_gather(table_hbm, idx0, bufs[b % _NB], b, gsems[b % _NB])

    def run_group(h, parity, g):
        row0 = base_b + g * _G
        idx_cur = idxs[parity]
        out_v = outs[parity]

        if parity == 0:
            pltpu.async_copy(
                ids_hbm.at[pl.ds((row0 + _G) * _S, _G * _S)],
                idxs[1], isems[1])
        else:
            @pl.when(h < half - 1)
            def _():
                pltpu.async_copy(
                    ids_hbm.at[pl.ds((row0 + _G) * _S, _G * _S)],
                    idxs[0], isems[0])

        @pl.when(h > 0)
        def _():
            drain_out(parity)

        for b in range(_G):
            nxt = b + _NB - 1
            if nxt < _G:
                _fire_gather(table_hbm, idx_cur, bufs[nxt % _NB], nxt,
                             gsems[nxt % _NB])
            else:
                if parity == 0:
                    if nxt == _G:
                        wait_idx(1)
                    _fire_gather(table_hbm, idxs[1], bufs[nxt % _NB],
                                 nxt - _G, gsems[nxt % _NB])
                else:
                    @pl.when(h < half - 1)
                    def _():
                        if nxt == _G:
                            wait_idx(0)
                        _fire_gather(table_hbm, idxs[0], bufs[nxt % _NB],
                                     nxt - _G, gsems[nxt % _NB])
            wait_buf(b % _NB)
            _reduce_mean(bufs[b % _NB], out_v, b)

        pltpu.async_copy(out_v, out_hbm.at[pl.ds(row0, _G)], osems[parity])

    def pair(h, carry):
        run_group(h, 0, 2 * h)
        run_group(h, 1, 2 * h + 1)
        return carry

    lax.fori_loop(0, half, pair, 0)
    drain_out(0)
    drain_out(1)


_emb_mean = functools.partial(
    pl.kernel,
    mesh=plsc.VectorSubcoreMesh(core_axis_name="c", subcore_axis_name="s"),
    out_type=jax.ShapeDtypeStruct((_B, _D), jnp.float32),
    scratch_types=[
        pltpu.VMEM((_G * _S,), jnp.int32),
        pltpu.VMEM((_G * _S,), jnp.int32),
        pltpu.VMEM((_S, _W), jnp.int32),
        pltpu.VMEM((_S, _W), jnp.int32),
        pltpu.VMEM((_S, _W), jnp.int32),
        pltpu.VMEM((_S, _W), jnp.int32),
        pltpu.VMEM((_S, _W), jnp.int32),
        pltpu.VMEM((_S, _W), jnp.int32),
        pltpu.VMEM((_S, _W), jnp.int32),
        pltpu.VMEM((_S, _W), jnp.int32),
        pltpu.VMEM((_G, _D), jnp.float32),
        pltpu.VMEM((_G, _D), jnp.float32),
        pltpu.SemaphoreType.DMA,
        pltpu.SemaphoreType.DMA,
        pltpu.SemaphoreType.DMA,
        pltpu.SemaphoreType.DMA,
        pltpu.SemaphoreType.DMA,
        pltpu.SemaphoreType.DMA,
        pltpu.SemaphoreType.DMA,
        pltpu.SemaphoreType.DMA,
        pltpu.SemaphoreType.DMA,
        pltpu.SemaphoreType.DMA,
        pltpu.SemaphoreType.DMA,
        pltpu.SemaphoreType.DMA,
    ],
    compiler_params=pltpu.CompilerParams(
        needs_layout_passes=False, use_tc_tiling_on_sc=False),
)(_emb_mean_body)


def _pack_table(table):
    tb = lax.bitcast_convert_type(table.astype(jnp.bfloat16), jnp.uint16)
    tb = tb.reshape(_VOCAB, _NCH, 2, _LANES).astype(jnp.uint32)
    w = tb[:, :, 0, :] | (tb[:, :, 1, :] << 16)
    return lax.bitcast_convert_type(w, jnp.int32).reshape(_VOCAB, _W)


@jax.jit
def kernel(input_ids, attention_mask, table):
    del attention_mask
    ids_flat = input_ids.reshape(-1).astype(jnp.int32) ^ jnp.int32(0)
    return _emb_mean(ids_flat, _pack_table(table))


# speedup vs baseline: 1.0265x; 1.0265x over previous
import functools

import jax
import jax.numpy as jnp
from jax import lax
from jax.experimental import pallas as pl
from jax.experimental.pallas import tpu as pltpu
from jax.experimental.pallas import tpu_sc as plsc

_VOCAB = 100000
_D = 128
_B = 16384
_S = 200
_W = _D // 2

_NC = 2
_NS = 16
_NW = _NC * _NS
_BPW = _B // _NW
_G = 8
_NGRP = _BPW // _G
_NB = 4
_RU = 4
_LANES = 16
_NCH = _D // (2 * _LANES)


def _fire_gather(table_hbm, idx_v, buf, b, sem):
    h0 = pltpu.async_copy(
        table_hbm.at[idx_v.at[pl.ds(b * _S, 128)]],
        buf.at[pl.ds(0, 128)], sem)
    h1 = pltpu.async_copy(
        table_hbm.at[idx_v.at[pl.ds(b * _S + 128, _S - 128)]],
        buf.at[pl.ds(128, _S - 128)], sem)
    return (h0, h1)


def _reduce_mean(buf, out_v, b):
    def body(r, accs):
        accs = list(accs)
        for u in range(_RU // 2):
            row = r * _RU + 2 * u
            for c in range(_NCH):
                w0 = buf[row, pl.ds(c * _LANES, _LANES)]
                w1 = buf[row + 1, pl.ds(c * _LANES, _LANES)]
                s = (plsc.bitcast(w0, jnp.bfloat16)
                     + plsc.bitcast(w1, jnp.bfloat16))
                a, bb = plsc.unpack(s, format=plsc.PackFormat.INTERLEAVED)
                accs[2 * c] = accs[2 * c] + a
                accs[2 * c + 1] = accs[2 * c + 1] + bb
        return tuple(accs)

    zero = jnp.zeros((_LANES,), jnp.float32)
    accs = lax.fori_loop(0, _S // _RU, body, (zero,) * (2 * _NCH))
    inv = jnp.float32(1.0 / _S)
    for c in range(_NCH):
        out_v[b, pl.ds(c * 2 * _LANES, _LANES)] = accs[2 * c] * inv
        out_v[b, pl.ds(c * 2 * _LANES + _LANES, _LANES)] = accs[2 * c + 1] * inv


def _emb_mean_body(ids_hbm, table_hbm, out_hbm,
                   idx0, idx1, rows0, rows1, rows2, rows3,
                   outv0, outv1,
                   semg0, semg1, semg2, semg3,
                   semi0, semi1, semo0, semo1):
    wid = lax.axis_index("s") * _NC + lax.axis_index("c")
    base_b = wid * _BPW
    bufs = (rows0, rows1, rows2, rows3)
    gsems = (semg0, semg1, semg2, semg3)
    idxs = (idx0, idx1)
    isems = (semi0, semi1)
    outs = (outv0, outv1)
    osems = (semo0, semo1)
    half = _NGRP // 2

    def wait_buf(k):
        pltpu.make_async_copy(table_hbm.at[pl.ds(0, _S)], bufs[k],
                              gsems[k]).wait()

    def wait_idx(p):
        pltpu.make_async_copy(ids_hbm.at[pl.ds(base_b * _S, _G * _S)],
                              idxs[p], isems[p]).wait()

    def drain_out(p):
        pltpu.make_async_copy(outs[p], out_hbm.at[pl.ds(base_b, _G)],
                              osems[p]).wait()

    pltpu.sync_copy(ids_hbm.at[pl.ds(base_b * _S, _G * _S)], idx0)
    for b in range(_NB - 1):
        _fire_gather(table_hbm, idx0, bufs[b % _NB], b, gsems[b % _NB])

    def run_group(h, parity, g):
        row0 = base_b + g * _G
        idx_cur = idxs[parity]
        out_v = outs[parity]

        if parity == 0:
            pltpu.async_copy(
                ids_hbm.at[pl.ds((row0 + _G) * _S, _G * _S)],
                idxs[1], isems[1])
        else:
            @pl.when(h < half - 1)
            def _():
                pltpu.async_copy(
                    ids_hbm.at[pl.ds((row0 + _G) * _S, _G * _S)],
                    idxs[0], isems[0])

        @pl.when(h > 0)
        def _():
            drain_out(parity)

        for b in range(_G):
            nxt = b + _NB - 1
            if nxt < _G:
                _fire_gather(table_hbm, idx_cur, bufs[nxt % _NB], nxt,
                             gsems[nxt % _NB])
            else:
                if parity == 0:
                    if nxt == _G:
                        wait_idx(1)
                    _fire_gather(table_hbm, idxs[1], bufs[nxt % _NB],
                                 nxt - _G, gsems[nxt % _NB])
                else:
                    @pl.when(h < half - 1)
                    def _():
                        if nxt == _G:
                            wait_idx(0)
                        _fire_gather(table_hbm, idxs[0], bufs[nxt % _NB],
                                     nxt - _G, gsems[nxt % _NB])
            wait_buf(b % _NB)
            _reduce_mean(bufs[b % _NB], out_v, b)

        pltpu.async_copy(out_v, out_hbm.at[pl.ds(row0, _G)], osems[parity])

    def pair(h, carry):
        run_group(h, 0, 2 * h)
        run_group(h, 1, 2 * h + 1)
        return carry

    lax.fori_loop(0, half, pair, 0)
    drain_out(0)
    drain_out(1)


_emb_mean = functools.partial(
    pl.kernel,
    mesh=plsc.VectorSubcoreMesh(core_axis_name="c", subcore_axis_name="s"),
    out_type=jax.ShapeDtypeStruct((_B, _D), jnp.float32),
    scratch_types=[
        pltpu.VMEM((_G * _S,), jnp.int32),
        pltpu.VMEM((_G * _S,), jnp.int32),
        pltpu.VMEM((_S, _W), jnp.int32),
        pltpu.VMEM((_S, _W), jnp.int32),
        pltpu.VMEM((_S, _W), jnp.int32),
        pltpu.VMEM((_S, _W), jnp.int32),
        pltpu.VMEM((_G, _D), jnp.float32),
        pltpu.VMEM((_G, _D), jnp.float32),
        pltpu.SemaphoreType.DMA,
        pltpu.SemaphoreType.DMA,
        pltpu.SemaphoreType.DMA,
        pltpu.SemaphoreType.DMA,
        pltpu.SemaphoreType.DMA,
        pltpu.SemaphoreType.DMA,
        pltpu.SemaphoreType.DMA,
        pltpu.SemaphoreType.DMA,
    ],
    compiler_params=pltpu.CompilerParams(
        needs_layout_passes=False, use_tc_tiling_on_sc=False),
)(_emb_mean_body)


def _pack_table(table):
    tb = lax.bitcast_convert_type(table.astype(jnp.bfloat16), jnp.uint16)
    tb = tb.reshape(_VOCAB, _NCH, 2, _LANES).astype(jnp.uint32)
    w = tb[:, :, 0, :] | (tb[:, :, 1, :] << 16)
    return lax.bitcast_convert_type(w, jnp.int32).reshape(_VOCAB, _W)


@jax.jit
def kernel(input_ids, attention_mask, table):
    del attention_mask
    ids_flat = input_ids.reshape(-1).astype(jnp.int32) ^ jnp.int32(0)
    return _emb_mean(ids_flat, _pack_table(table))


# speedup vs baseline: 1.0286x; 1.0021x over previous
import functools

import jax
import jax.numpy as jnp
from jax import lax
from jax.experimental import pallas as pl
from jax.experimental.pallas import tpu as pltpu
from jax.experimental.pallas import tpu_sc as plsc

_VOCAB = 100000
_D = 128
_B = 16384
_S = 200
_W = _D // 2

_NC = 2
_NS = 16
_NW = _NC * _NS
_BPW = _B // _NW
_G = 8
_NGRP = _BPW // _G
_NB = 4
_RU = 8
_LANES = 16
_NCH = _D // (2 * _LANES)


def _fire_gather(table_hbm, idx_v, buf, b, sem):
    h0 = pltpu.async_copy(
        table_hbm.at[idx_v.at[pl.ds(b * _S, 128)]],
        buf.at[pl.ds(0, 128)], sem)
    h1 = pltpu.async_copy(
        table_hbm.at[idx_v.at[pl.ds(b * _S + 128, _S - 128)]],
        buf.at[pl.ds(128, _S - 128)], sem)
    return (h0, h1)


def _reduce_mean(buf, out_v, b):
    def body(r, accs):
        accs = list(accs)
        for u in range(_RU // 2):
            row = r * _RU + 2 * u
            for c in range(_NCH):
                w0 = buf[row, pl.ds(c * _LANES, _LANES)]
                w1 = buf[row + 1, pl.ds(c * _LANES, _LANES)]
                s = (plsc.bitcast(w0, jnp.bfloat16)
                     + plsc.bitcast(w1, jnp.bfloat16))
                a, bb = plsc.unpack(s, format=plsc.PackFormat.INTERLEAVED)
                accs[2 * c] = accs[2 * c] + a
                accs[2 * c + 1] = accs[2 * c + 1] + bb
        return tuple(accs)

    zero = jnp.zeros((_LANES,), jnp.float32)
    accs = lax.fori_loop(0, _S // _RU, body, (zero,) * (2 * _NCH))
    inv = jnp.float32(1.0 / _S)
    for c in range(_NCH):
        out_v[b, pl.ds(c * 2 * _LANES, _LANES)] = accs[2 * c] * inv
        out_v[b, pl.ds(c * 2 * _LANES + _LANES, _LANES)] = accs[2 * c + 1] * inv


def _emb_mean_body(ids_hbm, table_hbm, out_hbm,
                   idx0, idx1, rows0, rows1, rows2, rows3,
                   outv0, outv1,
                   semg0, semg1, semg2, semg3,
                   semi0, semi1, semo0, semo1):
    wid = lax.axis_index("s") * _NC + lax.axis_index("c")
    base_b = wid * _BPW
    bufs = (rows0, rows1, rows2, rows3)
    gsems = (semg0, semg1, semg2, semg3)
    idxs = (idx0, idx1)
    isems = (semi0, semi1)
    outs = (outv0, outv1)
    osems = (semo0, semo1)
    half = _NGRP // 2

    def wait_buf(k):
        pltpu.make_async_copy(table_hbm.at[pl.ds(0, _S)], bufs[k],
                              gsems[k]).wait()

    def wait_idx(p):
        pltpu.make_async_copy(ids_hbm.at[pl.ds(base_b * _S, _G * _S)],
                              idxs[p], isems[p]).wait()

    def drain_out(p):
        pltpu.make_async_copy(outs[p], out_hbm.at[pl.ds(base_b, _G)],
                              osems[p]).wait()

    pltpu.sync_copy(ids_hbm.at[pl.ds(base_b * _S, _G * _S)], idx0)
    for b in range(_NB - 1):
        _fire_gather(table_hbm, idx0, bufs[b % _NB], b, gsems[b % _NB])

    def run_group(h, parity, g):
        row0 = base_b + g * _G
        idx_cur = idxs[parity]
        out_v = outs[parity]

        if parity == 0:
            pltpu.async_copy(
                ids_hbm.at[pl.ds((row0 + _G) * _S, _G * _S)],
                idxs[1], isems[1])
        else:
            @pl.when(h < half - 1)
            def _():
                pltpu.async_copy(
                    ids_hbm.at[pl.ds((row0 + _G) * _S, _G * _S)],
                    idxs[0], isems[0])

        @pl.when(h > 0)
        def _():
            drain_out(parity)

        for b in range(_G):
            nxt = b + _NB - 1
            if nxt < _G:
                _fire_gather(table_hbm, idx_cur, bufs[nxt % _NB], nxt,
                             gsems[nxt % _NB])
            else:
                if parity == 0:
                    if nxt == _G:
                        wait_idx(1)
                    _fire_gather(table_hbm, idxs[1], bufs[nxt % _NB],
                                 nxt - _G, gsems[nxt % _NB])
                else:
                    @pl.when(h < half - 1)
                    def _():
                        if nxt == _G:
                            wait_idx(0)
                        _fire_gather(table_hbm, idxs[0], bufs[nxt % _NB],
                                     nxt - _G, gsems[nxt % _NB])
            wait_buf(b % _NB)
            _reduce_mean(bufs[b % _NB], out_v, b)

        pltpu.async_copy(out_v, out_hbm.at[pl.ds(row0, _G)], osems[parity])

    def pair(h, carry):
        run_group(h, 0, 2 * h)
        run_group(h, 1, 2 * h + 1)
        return carry

    lax.fori_loop(0, half, pair, 0)
    drain_out(0)
    drain_out(1)


_emb_mean = functools.partial(
    pl.kernel,
    mesh=plsc.VectorSubcoreMesh(core_axis_name="c", subcore_axis_name="s"),
    out_type=jax.ShapeDtypeStruct((_B, _D), jnp.float32),
    scratch_types=[
        pltpu.VMEM((_G * _S,), jnp.int32),
        pltpu.VMEM((_G * _S,), jnp.int32),
        pltpu.VMEM((_S, _W), jnp.int32),
        pltpu.VMEM((_S, _W), jnp.int32),
        pltpu.VMEM((_S, _W), jnp.int32),
        pltpu.VMEM((_S, _W), jnp.int32),
        pltpu.VMEM((_G, _D), jnp.float32),
        pltpu.VMEM((_G, _D), jnp.float32),
        pltpu.SemaphoreType.DMA,
        pltpu.SemaphoreType.DMA,
        pltpu.SemaphoreType.DMA,
        pltpu.SemaphoreType.DMA,
        pltpu.SemaphoreType.DMA,
        pltpu.SemaphoreType.DMA,
        pltpu.SemaphoreType.DMA,
        pltpu.SemaphoreType.DMA,
    ],
    compiler_params=pltpu.CompilerParams(
        needs_layout_passes=False, use_tc_tiling_on_sc=False),
)(_emb_mean_body)


def _pack_table(table):
    tb = lax.bitcast_convert_type(table.astype(jnp.bfloat16), jnp.uint16)
    tb = tb.reshape(_VOCAB, _NCH, 2, _LANES).astype(jnp.uint32)
    w = tb[:, :, 0, :] | (tb[:, :, 1, :] << 16)
    return lax.bitcast_convert_type(w, jnp.int32).reshape(_VOCAB, _W)


@jax.jit
def kernel(input_ids, attention_mask, table):
    del attention_mask
    ids_flat = input_ids.reshape(-1).astype(jnp.int32) ^ jnp.int32(0)
    return _emb_mean(ids_flat, _pack_table(table))


# speedup vs baseline: 1.0330x; 1.0043x over previous
import functools

import jax
import jax.numpy as jnp
from jax import lax
from jax.experimental import pallas as pl
from jax.experimental.pallas import tpu as pltpu
from jax.experimental.pallas import tpu_sc as plsc

_VOCAB = 100000
_D = 128
_B = 16384
_S = 200
_W = _D // 2

_NC = 2
_NS = 16
_NW = _NC * _NS
_BPW = _B // _NW
_G = 8
_NGRP = _BPW // _G
_NB = 4
_RU = 4
_LANES = 16
_NCH = _D // (2 * _LANES)


def _fire_gather(table_hbm, idx_v, buf, b, sem):
    h0 = pltpu.async_copy(
        table_hbm.at[idx_v.at[pl.ds(b * _S, 128)]],
        buf.at[pl.ds(0, 128)], sem)
    h1 = pltpu.async_copy(
        table_hbm.at[idx_v.at[pl.ds(b * _S + 128, _S - 128)]],
        buf.at[pl.ds(128, _S - 128)], sem)
    return (h0, h1)


def _reduce_mean(buf, out_v, b):
    def body(r, accs):
        accs = list(accs)
        for u in range(_RU // 2):
            row = r * _RU + 2 * u
            for c in range(_NCH):
                w0 = buf[row, pl.ds(c * _LANES, _LANES)]
                w1 = buf[row + 1, pl.ds(c * _LANES, _LANES)]
                s = (plsc.bitcast(w0, jnp.bfloat16)
                     + plsc.bitcast(w1, jnp.bfloat16))
                a, bb = plsc.unpack(s, format=plsc.PackFormat.INTERLEAVED)
                accs[2 * c] = accs[2 * c] + a
                accs[2 * c + 1] = accs[2 * c + 1] + bb
        return tuple(accs)

    zero = jnp.zeros((_LANES,), jnp.float32)
    accs = lax.fori_loop(0, _S // _RU, body, (zero,) * (2 * _NCH))
    inv = jnp.float32(1.0 / _S)
    for c in range(_NCH):
        out_v[b, pl.ds(c * 2 * _LANES, _LANES)] = accs[2 * c] * inv
        out_v[b, pl.ds(c * 2 * _LANES + _LANES, _LANES)] = accs[2 * c + 1] * inv


def _emb_mean_body(ids_hbm, table_hbm, out_hbm,
                   idx0, idx1, rows0, rows1, rows2, rows3,
                   outv0, outv1,
                   semg0, semg1, semg2, semg3,
                   semi0, semi1, semo0, semo1):
    wid = lax.axis_index("s") * _NC + lax.axis_index("c")
    base_b = wid * _BPW
    bufs = (rows0, rows1, rows2, rows3)
    gsems = (semg0, semg1, semg2, semg3)
    idxs = (idx0, idx1)
    isems = (semi0, semi1)
    outs = (outv0, outv1)
    osems = (semo0, semo1)
    half = _NGRP // 2

    def wait_buf(k):
        pltpu.make_async_copy(table_hbm.at[pl.ds(0, _S)], bufs[k],
                              gsems[k]).wait()

    def wait_idx(p):
        pltpu.make_async_copy(ids_hbm.at[pl.ds(base_b * _S, _G * _S)],
                              idxs[p], isems[p]).wait()

    def drain_out(p):
        pltpu.make_async_copy(outs[p], out_hbm.at[pl.ds(base_b, _G)],
                              osems[p]).wait()

    pltpu.sync_copy(ids_hbm.at[pl.ds(base_b * _S, _G * _S)], idx0)
    for b in range(_NB - 1):
        _fire_gather(table_hbm, idx0, bufs[b % _NB], b, gsems[b % _NB])

    def run_group(h, parity, g):
        row0 = base_b + g * _G
        idx_cur = idxs[parity]
        out_v = outs[parity]

        if parity == 0:
            pltpu.async_copy(
                ids_hbm.at[pl.ds((row0 + _G) * _S, _G * _S)],
                idxs[1], isems[1])
        else:
            @pl.when(h < half - 1)
            def _():
                pltpu.async_copy(
                    ids_hbm.at[pl.ds((row0 + _G) * _S, _G * _S)],
                    idxs[0], isems[0])

        @pl.when(h > 0)
        def _():
            drain_out(parity)

        for b in range(_G):
            nxt = b + _NB - 1
            if nxt < _G:
                _fire_gather(table_hbm, idx_cur, bufs[nxt % _NB], nxt,
                             gsems[nxt % _NB])
            else:
                if parity == 0:
                    if nxt == _G:
                        wait_idx(1)
                    _fire_gather(table_hbm, idxs[1], bufs[nxt % _NB],
                                 nxt - _G, gsems[nxt % _NB])
                else:
                    @pl.when(h < half - 1)
                    def _():
                        if nxt == _G:
                            wait_idx(0)
                        _fire_gather(table_hbm, idxs[0], bufs[nxt % _NB],
                                     nxt - _G, gsems[nxt % _NB])
            wait_buf(b % _NB)
            _reduce_mean(bufs[b % _NB], out_v, b)

        pltpu.async_copy(out_v, out_hbm.at[pl.ds(row0, _G)], osems[parity])

    def pair(h, carry):
        run_group(h, 0, 2 * h)
        run_group(h, 1, 2 * h + 1)
        return carry

    lax.fori_loop(0, half, pair, 0)
    drain_out(0)
    drain_out(1)


_emb_mean = functools.partial(
    pl.kernel,
    mesh=plsc.VectorSubcoreMesh(core_axis_name="c", subcore_axis_name="s"),
    out_type=jax.ShapeDtypeStruct((_B, _D), jnp.float32),
    scratch_types=[
        pltpu.VMEM((_G * _S,), jnp.int32),
        pltpu.VMEM((_G * _S,), jnp.int32),
        pltpu.VMEM((_S, _W), jnp.int32),
        pltpu.VMEM((_S, _W), jnp.int32),
        pltpu.VMEM((_S, _W), jnp.int32),
        pltpu.VMEM((_S, _W), jnp.int32),
        pltpu.VMEM((_G, _D), jnp.float32),
        pltpu.VMEM((_G, _D), jnp.float32),
        pltpu.SemaphoreType.DMA,
        pltpu.SemaphoreType.DMA,
        pltpu.SemaphoreType.DMA,
        pltpu.SemaphoreType.DMA,
        pltpu.SemaphoreType.DMA,
        pltpu.SemaphoreType.DMA,
        pltpu.SemaphoreType.DMA,
        pltpu.SemaphoreType.DMA,
    ],
    compiler_params=pltpu.CompilerParams(
        needs_layout_passes=False, use_tc_tiling_on_sc=False),
)(_emb_mean_body)


def _pack_table(table):
    tb = lax.bitcast_convert_type(table.astype(jnp.bfloat16), jnp.uint16)
    tb = tb.reshape(_VOCAB, _NCH, 2, _LANES).astype(jnp.uint32)
    w = tb[:, :, 0, :] | (tb[:, :, 1, :] << 16)
    return lax.bitcast_convert_type(w, jnp.int32).reshape(_VOCAB, _W)


@jax.jit
def kernel(input_ids, attention_mask, table):
    del attention_mask
    ids_flat = input_ids.reshape(-1).astype(jnp.int32)
    return _emb_mean(ids_flat, _pack_table(table))
